# baseline (speedup 1.0000x reference)
import functools

import jax
import jax.numpy as jnp
from jax import lax
from jax.experimental import pallas as pl
from jax.experimental.pallas import tpu as pltpu
from jax.experimental.pallas import tpu_sc as plsc

_CHUNK = 128


def kernel(ids, E_target, E_context):
    B = ids.shape[0]
    D = E_target.shape[1]
    info = plsc.get_sparse_core_info()
    NC, NS = info.num_cores, info.num_subcores
    NW = NC * NS
    n_chunks = B // _CHUNK
    chunks_per_w = n_chunks // NW

    ids2 = ids.astype(jnp.int32).reshape(n_chunks, _CHUNK)
    mesh = plsc.VectorSubcoreMesh(core_axis_name="c", subcore_axis_name="s")

    @functools.partial(
        pl.kernel,
        mesh=mesh,
        out_type=jax.ShapeDtypeStruct((B, 2 * D), jnp.float32),
        compiler_params=pltpu.CompilerParams(use_tc_tiling_on_sc=False),
        scratch_types=[
            pltpu.VMEM((chunks_per_w, _CHUNK), jnp.int32),
            pltpu.VMEM((chunks_per_w, _CHUNK, D), jnp.float32),
            pltpu.VMEM((chunks_per_w, _CHUNK, D), jnp.float32),
            pltpu.SemaphoreType.DMA,
            pltpu.SemaphoreType.DMA,
        ],
    )
    def _k(ids_hbm, tgt_hbm, ctx_hbm, out_hbm, idx_v, buf_t, buf_c, gsem, wsem):
        wid = lax.axis_index("s") * NC + lax.axis_index("c")
        c0 = wid * chunks_per_w
        pltpu.sync_copy(ids_hbm.at[pl.ds(c0, chunks_per_w)], idx_v)
        gathers = []
        for j in range(chunks_per_w):
            gathers.append(
                pltpu.async_copy(tgt_hbm.at[idx_v.at[j]], buf_t.at[j], gsem))
            gathers.append(
                pltpu.async_copy(ctx_hbm.at[idx_v.at[j]], buf_c.at[j], gsem))
        for g in gathers:
            g.wait()
        writes = []
        for j in range(chunks_per_w):
            row0 = (c0 + j) * _CHUNK
            writes.append(pltpu.async_copy(
                buf_t.at[j], out_hbm.at[pl.ds(row0, _CHUNK), pl.ds(0, D)], wsem))
            writes.append(pltpu.async_copy(
                buf_c.at[j], out_hbm.at[pl.ds(row0, _CHUNK), pl.ds(D, D)], wsem))
        for w in writes:
            w.wait()

    return _k(ids2, E_target, E_context)


# speedup vs baseline: 4.6280x; 4.6280x over previous
import functools

import jax
import jax.numpy as jnp
from jax import lax
from jax.experimental import pallas as pl
from jax.experimental.pallas import tpu as pltpu
from jax.experimental.pallas import tpu_sc as plsc

_NBUF = 6


def kernel(ids, E_target, E_context):
    B = ids.shape[0]
    V, D = E_target.shape
    G = D // 8
    info = plsc.get_sparse_core_info()
    NC, NS = info.num_cores, info.num_subcores
    NW = NC * NS
    per_w = B // NW

    tt = E_target.T
    ct = E_context.T
    ids32 = ids.astype(jnp.int32)
    mesh = plsc.VectorSubcoreMesh(core_axis_name="c", subcore_axis_name="s")

    @functools.partial(
        pl.kernel,
        mesh=mesh,
        out_type=jax.ShapeDtypeStruct((2 * D, B), jnp.float32),
        compiler_params=pltpu.CompilerParams(needs_layout_passes=False),
        scratch_types=[
            pltpu.VMEM((per_w + 16,), jnp.int32),
            pltpu.VMEM((_NBUF, 2, G, 8, 128), jnp.float32),
            pltpu.VMEM((2 * D, per_w), jnp.float32),
            pltpu.SemaphoreType.DMA,
            pltpu.SemaphoreType.DMA,
        ],
    )
    def _k(ids_hbm, tt_hbm, ct_hbm, out_hbm, idsv, tb, blk, gsem, wsem):
        wid = lax.axis_index("s") * NC + lax.axis_index("c")
        base = wid * per_w
        pltpu.sync_copy(ids_hbm.at[pl.ds(base, per_w)],
                        idsv.at[pl.ds(0, per_w)])
        lane16 = lax.iota(jnp.int32, 16)
        zeros16 = lane16 * 0

        def col_of(i):
            s = idsv[pl.ds(i, 16)][0]
            k0 = pl.multiple_of((s >> 7) << 7, 128)
            return k0, s - k0

        def fire(i):
            slot = lax.rem(i, _NBUF)
            k0, _ = col_of(i)
            for g in range(G):
                rows = pl.ds(8 * g, 8)
                pltpu.async_copy(
                    tt_hbm.at[rows, pl.ds(k0, 128)], tb.at[slot, 0, g], gsem)
                pltpu.async_copy(
                    ct_hbm.at[rows, pl.ds(k0, 128)], tb.at[slot, 1, g], gsem)

        def extract(i):
            slot = lax.rem(i, _NBUF)
            _, l = col_of(i)
            for g in range(2 * G):
                pltpu.make_async_copy(
                    tt_hbm.at[pl.ds(0, 8), pl.ds(0, 128)],
                    tb.at[slot, 0, 0], gsem).wait()
            lvec = zeros16 + l
            ivec = zeros16 + i
            for t in range(2):
                tsel = zeros16 + t
                for h in range(D // 16):
                    gvec = (lane16 + h * 16) >> 3
                    rvec = lane16 & 7
                    v = plsc.load_gather(tb, [zeros16 + slot, tsel, gvec,
                                              rvec, lvec])
                    plsc.store_scatter(
                        blk, [lane16 + (t * D + h * 16), ivec], v)

        def main_body(i, _):
            fire(i)

            @pl.when(i >= _NBUF - 1)
            def _():
                extract(i - (_NBUF - 1))

            return 0

        lax.fori_loop(0, per_w, main_body, 0)

        def tail_body(i, _):
            extract(i)
            return 0

        lax.fori_loop(per_w - (_NBUF - 1), per_w, tail_body, 0)
        pltpu.sync_copy(blk, out_hbm.at[pl.ds(0, 2 * D), pl.ds(base, per_w)])

    return _k(ids32, tt, ct).T


# speedup vs baseline: 4.6297x; 1.0004x over previous
import functools

import jax
import jax.numpy as jnp
from jax import lax
from jax.experimental import pallas as pl
from jax.experimental.pallas import tpu as pltpu
from jax.experimental.pallas import tpu_sc as plsc

_NBUF = 8


def kernel(ids, E_target, E_context):
    B = ids.shape[0]
    V, D = E_target.shape
    G = D // 8
    info = plsc.get_sparse_core_info()
    NC, NS = info.num_cores, info.num_subcores
    NW = NC * NS
    per_w = B // NW

    tt = E_target.T
    ct = E_context.T
    ids32 = ids.astype(jnp.int32)
    mesh = plsc.VectorSubcoreMesh(core_axis_name="c", subcore_axis_name="s")

    @functools.partial(
        pl.kernel,
        mesh=mesh,
        out_type=jax.ShapeDtypeStruct((2 * D, B), jnp.float32),
        compiler_params=pltpu.CompilerParams(needs_layout_passes=False),
        scratch_types=[
            pltpu.VMEM((per_w + 16,), jnp.int32),
            pltpu.VMEM((_NBUF, 2, D, 128), jnp.float32),
            pltpu.VMEM((2 * D, per_w), jnp.float32),
            pltpu.SemaphoreType.DMA,
            pltpu.SemaphoreType.DMA,
        ],
    )
    def _k(ids_hbm, tt_hbm, ct_hbm, out_hbm, idsv, tb, blk, gsem, wsem):
        wid = lax.axis_index("s") * NC + lax.axis_index("c")
        base = wid * per_w
        pltpu.sync_copy(ids_hbm.at[pl.ds(base, per_w)],
                        idsv.at[pl.ds(0, per_w)])
        lane16 = lax.iota(jnp.int32, 16)
        zeros16 = lane16 * 0

        def col_of(i):
            s = idsv[pl.ds(i, 16)][0]
            k0 = pl.multiple_of((s >> 7) << 7, 128)
            return k0, s - k0

        def fire(i):
            slot = lax.rem(i, _NBUF)
            k0, _ = col_of(i)
            cols = pl.ds(k0, 128)
            pltpu.async_copy(tt_hbm.at[:, cols], tb.at[slot, 0], gsem)
            pltpu.async_copy(ct_hbm.at[:, cols], tb.at[slot, 1], gsem)

        def extract(i):
            slot = lax.rem(i, _NBUF)
            _, l = col_of(i)
            for t in range(2):
                pltpu.make_async_copy(
                    tt_hbm.at[pl.ds(0, D), pl.ds(0, 128)],
                    tb.at[slot, 0], gsem).wait()
            lvec = zeros16 + l
            ivec = zeros16 + i
            for t in range(2):
                tsel = zeros16 + t
                for h in range(D // 16):
                    cvec = lane16 + h * 16
                    v = plsc.load_gather(tb, [zeros16 + slot, tsel, cvec,
                                              lvec])
                    plsc.store_scatter(
                        blk, [lane16 + (t * D + h * 16), ivec], v)

        def main_body(i, _):
            fire(i)

            @pl.when(i >= _NBUF - 1)
            def _():
                extract(i - (_NBUF - 1))

            return 0

        lax.fori_loop(0, per_w, main_body, 0)

        def tail_body(i, _):
            extract(i)
            return 0

        lax.fori_loop(per_w - (_NBUF - 1), per_w, tail_body, 0)
        pltpu.sync_copy(blk, out_hbm.at[pl.ds(0, 2 * D), pl.ds(base, per_w)])

    return _k(ids32, tt, ct).T
